# Initial kernel scaffold; baseline (speedup 1.0000x reference)
import jax
import jax.numpy as jnp
from jax.experimental import pallas as pl


def kernel(embeddings, rand_proj, codebook, values):
    raise NotImplementedError("write your pallas kernel here")


# speedup vs baseline: 1.0799x; 1.0799x over previous
import jax
import jax.numpy as jnp
from jax.experimental import pallas as pl
from jax.experimental.pallas import tpu as pltpu

_B, _N, _D = 16, 196, 768
_H, _E, _K, _DM = 16, 64, 2048, 64
_BN = _B * _N
_T = 224
_PREC = jax.lax.Precision.DEFAULT


def _dkvb_block(emb_ref, proj_ref, c2t_ref, vals_ref, out_ref, esq_ref):
    @pl.when(pl.program_id(0) == 0)
    def _():
        esq_ref[...] = 0.25 * jnp.sum(c2t_ref[...] ** 2, axis=1)

    x = jnp.dot(emb_ref[...], proj_ref[...], precision=_PREC)
    acc = jnp.zeros((_T, _DM), jnp.float32)
    iota = jax.lax.broadcasted_iota(jnp.int32, (_T, _K), 1)
    for h in range(_H):
        xh = x[:, h * _E:(h + 1) * _E]
        score = jnp.dot(xh, c2t_ref[h], precision=_PREC) - esq_ref[h][None, :]
        mx = jnp.max(score, axis=1, keepdims=True)
        idx = jnp.min(jnp.where(score == mx, iota, _K), axis=1)
        onehot = (iota == idx[:, None]).astype(jnp.bfloat16)
        acc = acc + jnp.dot(onehot, vals_ref[h],
                            preferred_element_type=jnp.float32)
    out_ref[...] = acc * (1.0 / _H)


def kernel(embeddings, rand_proj, codebook, values):
    emb = embeddings.reshape(_BN, _D)
    proj = jnp.transpose(rand_proj, (1, 0, 2)).reshape(_D, _H * _E)
    c2t = jnp.transpose(codebook * 2.0, (0, 2, 1))
    vals = values.astype(jnp.bfloat16)

    out = pl.pallas_call(
        _dkvb_block,
        grid=(_BN // _T,),
        in_specs=[
            pl.BlockSpec((_T, _D), lambda i: (i, 0)),
            pl.BlockSpec((_D, _H * _E), lambda i: (0, 0)),
            pl.BlockSpec((_H, _E, _K), lambda i: (0, 0, 0)),
            pl.BlockSpec((_H, _K, _DM), lambda i: (0, 0, 0)),
        ],
        out_specs=pl.BlockSpec((_T, _DM), lambda i: (i, 0)),
        out_shape=jax.ShapeDtypeStruct((_BN, _DM), jnp.float32),
        scratch_shapes=[pltpu.VMEM((_H, _K), jnp.float32)],
        compiler_params=pltpu.CompilerParams(
            dimension_semantics=("arbitrary",)),
    )(emb, proj, c2t, vals)
    return out.reshape(_B, _N, _DM)


# speedup vs baseline: 1.5209x; 1.4084x over previous
import jax
import jax.numpy as jnp
from jax import lax
from jax.experimental import pallas as pl
from jax.experimental.pallas import tpu as pltpu
from jax.experimental.pallas import tpu_sc as plsc

_B, _N, _D = 16, 196, 768
_H, _E, _K, _DM = 16, 64, 2048, 64
_BN = _B * _N
_T = 224
_KT = 256
_PREC = jax.lax.Precision.DEFAULT

_NW = 32
_TPW = _BN // _NW


def _tc_block(emb_ref, proj_ref, c2t_ref, idx_ref, esq_ref):
    @pl.when(pl.program_id(0) == 0)
    def _():
        esq_ref[...] = 0.25 * jnp.sum(c2t_ref[...] ** 2, axis=1)

    x = jnp.dot(emb_ref[...], proj_ref[...], precision=_PREC)
    per_head = []
    for h in range(_H):
        xh = x[:, h * _E:(h + 1) * _E]
        dots2 = jnp.dot(xh, c2t_ref[h], precision=_PREC)
        best = dots2[:, 0:_KT] - esq_ref[h][None, 0:_KT]
        bestk = lax.broadcasted_iota(jnp.int32, (_T, _KT), 1)
        for kt in range(1, _K // _KT):
            tile = (dots2[:, kt * _KT:(kt + 1) * _KT]
                    - esq_ref[h][None, kt * _KT:(kt + 1) * _KT])
            ktile = lax.broadcasted_iota(jnp.int32, (_T, _KT), 1) + kt * _KT
            upd = tile > best
            best = jnp.where(upd, tile, best)
            bestk = jnp.where(upd, ktile, bestk)
        mx = jnp.max(best, axis=1, keepdims=True)
        cand = jnp.where(best == mx, bestk, _K)
        per_head.append(jnp.min(cand, axis=1) + h * _K)
    idx_all = jnp.stack(per_head, axis=0)
    idx_ref[...] = idx_all.T


_TPR = _TPW // 2


def _sc_gather_mean(vals_hbm, idx_hbm, out_hbm, idx_v, rows_v, out_v, sem):
    wid = lax.axis_index("s") * 2 + lax.axis_index("c")
    base = wid * _TPW
    pltpu.sync_copy(idx_hbm.at[pl.ds(base * _H, _TPW * _H)], idx_v)

    for r in range(2):
        pltpu.async_copy(
            vals_hbm.at[idx_v.at[pl.ds(r * _TPR * _H, _TPR * _H)]],
            rows_v, sem).wait()

        @pl.loop(0, _TPR)
        def _(t):
            for c in range(0, _DM, 16):
                acc = rows_v.at[t * _H, pl.ds(c, 16)][...]
                for h in range(1, _H):
                    acc = acc + rows_v.at[t * _H + h, pl.ds(c, 16)][...]
                out_v.at[pl.ds((r * _TPR + t) * _DM + c, 16)][...] = (
                    acc * (1.0 / _H))

    pltpu.sync_copy(out_v, out_hbm.at[pl.ds(base * _DM, _TPW * _DM)])


def kernel(embeddings, rand_proj, codebook, values):
    emb = embeddings.reshape(_BN, _D)
    proj = jnp.transpose(rand_proj, (1, 0, 2)).reshape(_D, _H * _E)
    c2t = jnp.transpose(codebook * 2.0, (0, 2, 1))

    idx = pl.pallas_call(
        _tc_block,
        grid=(_BN // _T,),
        in_specs=[
            pl.BlockSpec((_T, _D), lambda i: (i, 0)),
            pl.BlockSpec((_D, _H * _E), lambda i: (0, 0)),
            pl.BlockSpec((_H, _E, _K), lambda i: (0, 0, 0)),
        ],
        out_specs=pl.BlockSpec((_T, _H), lambda i: (i, 0)),
        out_shape=jax.ShapeDtypeStruct((_BN, _H), jnp.int32),
        scratch_shapes=[pltpu.VMEM((_H, _K), jnp.float32)],
        compiler_params=pltpu.CompilerParams(
            dimension_semantics=("arbitrary",)),
    )(emb, proj, c2t)

    vals_flat = jnp.pad(values.reshape(_H * _K, _DM), ((0, 0), (0, 128 - _DM)))
    idx_flat = idx.reshape(_BN * _H)
    mesh = plsc.VectorSubcoreMesh(core_axis_name="c", subcore_axis_name="s")
    sc = pl.kernel(
        _sc_gather_mean,
        mesh=mesh,
        out_type=jax.ShapeDtypeStruct((_BN * _DM,), jnp.float32),
        scratch_types=[
            pltpu.VMEM((_TPW * _H,), jnp.int32),
            pltpu.VMEM((_TPR * _H, 128), jnp.float32),
            pltpu.VMEM((_TPW * _DM,), jnp.float32),
            pltpu.SemaphoreType.DMA,
        ],
    )
    out = sc(vals_flat, idx_flat)
    return out.reshape(_B, _N, _DM)
